# baseline (speedup 1.0000x reference)
import functools

import jax
import jax.numpy as jnp
from jax import lax
from jax.experimental import pallas as pl
from jax.experimental.pallas import tpu as pltpu
from jax.experimental.pallas import tpu_sc as plsc

N = 10000
E = 160000
IN_DIM = 16
OUT_DIM = 16
EDGE_DIM = 16

NC = 2
NS = 16
NW = NC * NS
CHUNK = 128
CH_PER_W = 40
EPW = CHUNK * CH_PER_W
E_PAD = EPW * NW
N_SP = 10048
ZROWS = N_SP // NS
OROWS = N // NS


HALF = EPW // 2


def _sc_gather_body(node_hbm, idx_hbm, out_hbm, idx_a, idx_b, rows_v, xt_v, sem):
    c = lax.axis_index("c")
    s = lax.axis_index("s")
    wid = s * NC + c
    base = wid * EPW
    iot = lax.iota(jnp.int32, 16)
    pltpu.sync_copy(idx_hbm.at[pl.ds(base, HALF)], idx_a)
    pltpu.sync_copy(idx_hbm.at[pl.ds(base + HALF, HALF)], idx_b)
    for h, idx_h in ((0, idx_a), (1, idx_b)):
        pltpu.async_copy(node_hbm.at[idx_h], rows_v, sem).wait()

        def tb(g, carry):
            b16 = g * 16
            for j in range(16):
                col = plsc.load_gather(
                    rows_v, [iot + b16, jnp.full((16,), j, jnp.int32)]
                )
                xt_v[pl.ds(j * HALF + b16, 16)] = col
            return carry

        lax.fori_loop(0, HALF // 16, tb, 0)
        for j in range(16):
            pltpu.sync_copy(xt_v.at[pl.ds(j * HALF, HALF)],
                            out_hbm.at[j, pl.ds(base + h * HALF, HALF)])


@functools.lru_cache(maxsize=None)
def _sc_gather():
    mesh = plsc.VectorSubcoreMesh(
        core_axis_name="c", subcore_axis_name="s", num_cores=NC, num_subcores=NS
    )
    return pl.kernel(
        _sc_gather_body,
        out_type=jax.ShapeDtypeStruct((IN_DIM, E_PAD), jnp.float32),
        mesh=mesh,
        compiler_params=pltpu.CompilerParams(use_tc_tiling_on_sc=False,
                                             needs_layout_passes=False),
        scratch_types=[
            pltpu.VMEM((HALF,), jnp.int32),
            pltpu.VMEM((HALF,), jnp.int32),
            pltpu.VMEM((HALF, IN_DIM), jnp.float32),
            pltpu.VMEM((IN_DIM * HALF,), jnp.float32),
            pltpu.SemaphoreType.DMA,
        ],
    )


QTR = EPW // 4


def _sc_scatter_body(msg_hbm, idx_hbm, zeros_hbm, ones_hbm, agg_out, cnt_out,
                     idx_0, idx_1, idx_2, idx_3, val_v, mt_v, ones_v,
                     agg_sh, cnt_sh):
    c = lax.axis_index("c")
    s = lax.axis_index("s")
    wid = s * NC + c
    base = wid * EPW
    iot = lax.iota(jnp.int32, 16)
    idx_refs = (idx_0, idx_1, idx_2, idx_3)

    pltpu.sync_copy(zeros_hbm, agg_sh.at[pl.ds(s * ZROWS, ZROWS)])
    pltpu.sync_copy(zeros_hbm, cnt_sh.at[pl.ds(s * ZROWS, ZROWS)])
    for q in range(4):
        pltpu.sync_copy(idx_hbm.at[pl.ds(base + q * QTR, QTR)], idx_refs[q])
    pltpu.sync_copy(ones_hbm, ones_v)
    plsc.subcore_barrier()

    for q in range(4):
        for j in range(16):
            pltpu.sync_copy(msg_hbm.at[j, pl.ds(base + q * QTR, QTR)],
                            mt_v.at[pl.ds(j * QTR, QTR)])

        def tb(g, carry):
            b16 = g * 16
            for j in range(16):
                row = mt_v[pl.ds(j * QTR + b16, 16)]
                plsc.store_scatter(
                    val_v, [iot + b16, jnp.full((16,), j, jnp.int32)], row
                )
            return carry

        lax.fori_loop(0, QTR // 16, tb, 0)
        pltpu.sync_copy(val_v, agg_sh.at[idx_refs[q]], add=True)

    for q in range(4):
        pltpu.sync_copy(ones_v, cnt_sh.at[idx_refs[q]], add=True)
    plsc.subcore_barrier()

    pltpu.sync_copy(agg_sh.at[pl.ds(s * OROWS, OROWS)],
                    agg_out.at[c, pl.ds(s * OROWS, OROWS)])
    pltpu.sync_copy(cnt_sh.at[pl.ds(s * OROWS, OROWS)],
                    cnt_out.at[c, pl.ds(s * OROWS, OROWS)])


@functools.lru_cache(maxsize=None)
def _sc_scatter():
    mesh = plsc.VectorSubcoreMesh(
        core_axis_name="c", subcore_axis_name="s", num_cores=NC, num_subcores=NS
    )
    return pl.kernel(
        _sc_scatter_body,
        out_type=(
            jax.ShapeDtypeStruct((NC, N, OUT_DIM), jnp.float32),
            jax.ShapeDtypeStruct((NC, N, OUT_DIM), jnp.float32),
        ),
        mesh=mesh,
        compiler_params=pltpu.CompilerParams(use_tc_tiling_on_sc=False,
                                             needs_layout_passes=False),
        scratch_types=[
            pltpu.VMEM((QTR,), jnp.int32),
            pltpu.VMEM((QTR,), jnp.int32),
            pltpu.VMEM((QTR,), jnp.int32),
            pltpu.VMEM((QTR,), jnp.int32),
            pltpu.VMEM((QTR, OUT_DIM), jnp.float32),
            pltpu.VMEM((OUT_DIM * QTR,), jnp.float32),
            pltpu.VMEM((QTR, OUT_DIM), jnp.float32),
            pltpu.VMEM_SHARED((N_SP, OUT_DIM), jnp.float32),
            pltpu.VMEM_SHARED((N_SP, OUT_DIM), jnp.float32),
        ],
    )


PK = 128 // IN_DIM
MSG_BLK = 1280
KD = IN_DIM * OUT_DIM


def _msg_body(ef_ref, x_ref, wnt_ref, bt_ref, rt_ref, f_ref, out_ref):
    ef = ef_ref[...]
    x = x_ref[...]
    t = jnp.tanh(
        jnp.dot(wnt_ref[...], ef, preferred_element_type=jnp.float32)
        + bt_ref[...]
    )
    xb = jnp.dot(rt_ref[...], x, preferred_element_type=jnp.float32)
    out_ref[...] = jnp.dot(f_ref[...], xb * t,
                           preferred_element_type=jnp.float32)


def _msg_call(eft, xt, wnt, bt, rt, f):
    return pl.pallas_call(
        _msg_body,
        grid=(E // MSG_BLK,),
        in_specs=[
            pl.BlockSpec((EDGE_DIM, MSG_BLK), lambda i: (0, i)),
            pl.BlockSpec((IN_DIM, MSG_BLK), lambda i: (0, i)),
            pl.BlockSpec((KD, EDGE_DIM), lambda i: (0, 0)),
            pl.BlockSpec((KD, 1), lambda i: (0, 0)),
            pl.BlockSpec((KD, IN_DIM), lambda i: (0, 0)),
            pl.BlockSpec((OUT_DIM, KD), lambda i: (0, 0)),
        ],
        out_specs=pl.BlockSpec((OUT_DIM, MSG_BLK), lambda i: (0, i)),
        out_shape=jax.ShapeDtypeStruct((OUT_DIM, E_PAD), jnp.float32),
    )(eft, xt, wnt, bt, rt, f)


N8 = N // PK


def _final_body(nf_ref, agg_ref, cnt_ref, wr2_ref, m_ref, rb_ref, g_ref, b_ref,
                out_ref):
    nf = nf_ref[...]
    agg = agg_ref[0] + agg_ref[1]
    cnt = cnt_ref[0] + cnt_ref[1]
    agg = agg / jnp.maximum(cnt, 1.0)
    pre = (
        jnp.dot(nf, wr2_ref[...], preferred_element_type=jnp.float32)
        + agg
        + rb_ref[...]
    )
    csum = jnp.sum(pre, axis=0, keepdims=True)
    csq = jnp.sum(pre * pre, axis=0, keepdims=True)
    mu = jnp.dot(csum, m_ref[...], preferred_element_type=jnp.float32) / N
    musq = jnp.dot(csq, m_ref[...], preferred_element_type=jnp.float32) / N
    var = musq - mu * mu
    out = (pre - mu) / jnp.sqrt(var + 1e-5) * g_ref[...] + b_ref[...]
    out_ref[...] = jnp.where(out >= 0.0, out, 0.01 * out)


def _final_call(nf_pk, agg_pk, cnt_pk, wr2, m, rb, g, b):
    return pl.pallas_call(
        _final_body,
        out_shape=jax.ShapeDtypeStruct((N8, 128), jnp.float32),
    )(nf_pk, agg_pk, cnt_pk, wr2, m, rb, g, b)


def kernel(node_feat, edge_feat, edge_index, batch_index,
           num_sampled_nodes_per_hop, num_sampled_edges_per_hop,
           W_net, b_net, W_root, root_bias, bn_gamma, bn_beta):
    src = edge_index[0]
    dst = edge_index[1]
    pad = E_PAD - E
    src_p = jnp.pad(src, (0, pad))
    dst_p = jnp.pad(dst, (0, pad), constant_values=N)

    wnt = W_net.T
    bt = b_net.reshape(KD, 1)
    r0 = lax.broadcasted_iota(jnp.int32, (KD, IN_DIM), 0)
    r1 = lax.broadcasted_iota(jnp.int32, (KD, IN_DIM), 1)
    rt = (r0 // OUT_DIM == r1).astype(jnp.float32)
    f0 = lax.broadcasted_iota(jnp.int32, (OUT_DIM, KD), 0)
    f1 = lax.broadcasted_iota(jnp.int32, (OUT_DIM, KD), 1)
    f = (f1 % OUT_DIM == f0).astype(jnp.float32)

    xt = _sc_gather()(node_feat, src_p)
    msgt = _msg_call(edge_feat.T, xt, wnt, bt, rt, f)
    agg_parts, cnt_parts = _sc_scatter()(
        msgt, dst_p,
        jnp.zeros((ZROWS, OUT_DIM), jnp.float32),
        jnp.ones((QTR, OUT_DIM), jnp.float32),
    )

    wr2 = jnp.kron(jnp.eye(PK, dtype=jnp.float32), W_root)
    m0 = lax.broadcasted_iota(jnp.int32, (128, 128), 0)
    m1 = lax.broadcasted_iota(jnp.int32, (128, 128), 1)
    m = (m0 % OUT_DIM == m1 % OUT_DIM).astype(jnp.float32)
    out = _final_call(
        node_feat.reshape(N8, 128), agg_parts.reshape(NC, N8, 128),
        cnt_parts.reshape(NC, N8, 128), wr2, m,
        jnp.tile(root_bias, PK).reshape(1, 128),
        jnp.tile(bn_gamma, PK).reshape(1, 128),
        jnp.tile(bn_beta, PK).reshape(1, 128),
    )
    return (out.reshape(N, OUT_DIM), edge_index, edge_feat)


# speedup vs baseline: 1.2588x; 1.2588x over previous
import functools

import jax
import jax.numpy as jnp
from jax import lax
from jax.experimental import pallas as pl
from jax.experimental.pallas import tpu as pltpu
from jax.experimental.pallas import tpu_sc as plsc

N = 10000
E = 160000
IN_DIM = 16
OUT_DIM = 16
EDGE_DIM = 16

NC = 2
NS = 16
NW = NC * NS
EPW = 5120
E_PAD = EPW * NW
HALF = EPW // 2
N_SP = 10048
ZROWS = N_SP // NS
OROWS = N // NS

PK = 128 // IN_DIM
E8 = E // PK
E8_PAD = E_PAD // PK
N8 = N // PK
KD = PK * IN_DIM * OUT_DIM


def _sc_gather_body(node_hbm, idx_hbm, out_hbm, idx_v, rows_v, sem):
    c = lax.axis_index("c")
    s = lax.axis_index("s")
    wid = s * NC + c
    base = wid * EPW
    pltpu.sync_copy(idx_hbm.at[pl.ds(base, EPW)], idx_v)
    pltpu.async_copy(node_hbm.at[idx_v], rows_v, sem).wait()
    pltpu.sync_copy(rows_v, out_hbm.at[pl.ds(base, EPW)])


@functools.lru_cache(maxsize=None)
def _sc_gather():
    mesh = plsc.VectorSubcoreMesh(
        core_axis_name="c", subcore_axis_name="s", num_cores=NC, num_subcores=NS
    )
    return pl.kernel(
        _sc_gather_body,
        out_type=jax.ShapeDtypeStruct((E_PAD, IN_DIM), jnp.float32),
        mesh=mesh,
        compiler_params=pltpu.CompilerParams(use_tc_tiling_on_sc=False),
        scratch_types=[
            pltpu.VMEM((EPW,), jnp.int32),
            pltpu.VMEM((EPW, IN_DIM), jnp.float32),
            pltpu.SemaphoreType.DMA,
        ],
    )


def _sc_scatter_body(msg_hbm, idx_hbm, zeros_hbm, ones_hbm, agg_out, cnt_out,
                     idx_a, idx_b, val_v, ones_v, agg_sh, cnt_sh):
    c = lax.axis_index("c")
    s = lax.axis_index("s")
    wid = s * NC + c
    base = wid * EPW

    pltpu.sync_copy(zeros_hbm, agg_sh.at[pl.ds(s * ZROWS, ZROWS)])
    pltpu.sync_copy(zeros_hbm, cnt_sh.at[pl.ds(s * ZROWS, ZROWS)])
    pltpu.sync_copy(idx_hbm.at[pl.ds(base, HALF)], idx_a)
    pltpu.sync_copy(idx_hbm.at[pl.ds(base + HALF, HALF)], idx_b)
    pltpu.sync_copy(ones_hbm, ones_v)
    plsc.subcore_barrier()

    pltpu.sync_copy(msg_hbm.at[pl.ds(base, HALF)], val_v)
    pltpu.sync_copy(val_v, agg_sh.at[idx_a], add=True)
    pltpu.sync_copy(msg_hbm.at[pl.ds(base + HALF, HALF)], val_v)
    pltpu.sync_copy(val_v, agg_sh.at[idx_b], add=True)
    pltpu.sync_copy(ones_v, cnt_sh.at[idx_a], add=True)
    pltpu.sync_copy(ones_v, cnt_sh.at[idx_b], add=True)
    plsc.subcore_barrier()

    pltpu.sync_copy(agg_sh.at[pl.ds(s * OROWS, OROWS)],
                    agg_out.at[c, pl.ds(s * OROWS, OROWS)])
    pltpu.sync_copy(cnt_sh.at[pl.ds(s * OROWS, OROWS)],
                    cnt_out.at[c, pl.ds(s * OROWS, OROWS)])


@functools.lru_cache(maxsize=None)
def _sc_scatter():
    mesh = plsc.VectorSubcoreMesh(
        core_axis_name="c", subcore_axis_name="s", num_cores=NC, num_subcores=NS
    )
    return pl.kernel(
        _sc_scatter_body,
        out_type=(
            jax.ShapeDtypeStruct((NC, N, OUT_DIM), jnp.float32),
            jax.ShapeDtypeStruct((NC, N, OUT_DIM), jnp.float32),
        ),
        mesh=mesh,
        compiler_params=pltpu.CompilerParams(use_tc_tiling_on_sc=False),
        scratch_types=[
            pltpu.VMEM((HALF,), jnp.int32),
            pltpu.VMEM((HALF,), jnp.int32),
            pltpu.VMEM((HALF, OUT_DIM), jnp.float32),
            pltpu.VMEM((HALF, OUT_DIM), jnp.float32),
            pltpu.VMEM_SHARED((N_SP, OUT_DIM), jnp.float32),
            pltpu.VMEM_SHARED((N_SP, OUT_DIM), jnp.float32),
        ],
    )


MSG_BLK8 = 400


def _msg_body(ef_ref, x_ref, w2_ref, b2_ref, r2_ref, s2_ref, out_ref):
    ef = ef_ref[...]
    x = x_ref[...]
    t = jnp.tanh(
        jnp.dot(ef, w2_ref[...], preferred_element_type=jnp.float32) + b2_ref[...]
    )
    xb = jnp.dot(x, r2_ref[...], preferred_element_type=jnp.float32)
    out_ref[...] = jnp.dot(xb * t, s2_ref[...], preferred_element_type=jnp.float32)


def _msg_call(ef_pk, x_pk, w2, b2, r2, s2):
    return pl.pallas_call(
        _msg_body,
        grid=(E8 // MSG_BLK8,),
        in_specs=[
            pl.BlockSpec((MSG_BLK8, 128), lambda i: (i, 0)),
            pl.BlockSpec((MSG_BLK8, 128), lambda i: (i, 0)),
            pl.BlockSpec((128, KD), lambda i: (0, 0)),
            pl.BlockSpec((1, KD), lambda i: (0, 0)),
            pl.BlockSpec((128, KD), lambda i: (0, 0)),
            pl.BlockSpec((KD, 128), lambda i: (0, 0)),
        ],
        out_specs=pl.BlockSpec((MSG_BLK8, 128), lambda i: (i, 0)),
        out_shape=jax.ShapeDtypeStruct((E8_PAD, 128), jnp.float32),
    )(ef_pk, x_pk, w2, b2, r2, s2)


def _final_body(nf_ref, agg_ref, cnt_ref, wr2_ref, m_ref, rb_ref, g_ref, b_ref,
                out_ref):
    nf = nf_ref[...]
    agg = agg_ref[0] + agg_ref[1]
    cnt = cnt_ref[0] + cnt_ref[1]
    agg = agg / jnp.maximum(cnt, 1.0)
    pre = (
        jnp.dot(nf, wr2_ref[...], preferred_element_type=jnp.float32)
        + agg
        + rb_ref[...]
    )
    csum = jnp.sum(pre, axis=0, keepdims=True)
    csq = jnp.sum(pre * pre, axis=0, keepdims=True)
    mu = jnp.dot(csum, m_ref[...], preferred_element_type=jnp.float32) / N
    musq = jnp.dot(csq, m_ref[...], preferred_element_type=jnp.float32) / N
    var = musq - mu * mu
    out = (pre - mu) / jnp.sqrt(var + 1e-5) * g_ref[...] + b_ref[...]
    out_ref[...] = jnp.where(out >= 0.0, out, 0.01 * out)


def _final_call(nf_pk, agg_pk, cnt_pk, wr2, m, rb, g, b):
    return pl.pallas_call(
        _final_body,
        out_shape=jax.ShapeDtypeStruct((N8, 128), jnp.float32),
    )(nf_pk, agg_pk, cnt_pk, wr2, m, rb, g, b)


def kernel(node_feat, edge_feat, edge_index, batch_index,
           num_sampled_nodes_per_hop, num_sampled_edges_per_hop,
           W_net, b_net, W_root, root_bias, bn_gamma, bn_beta):
    src = edge_index[0]
    dst = edge_index[1]
    pad = E_PAD - E
    src_p = jnp.pad(src, (0, pad))
    dst_p = jnp.pad(dst, (0, pad), constant_values=N)

    eye8 = jnp.eye(PK, dtype=jnp.float32)
    w2 = jnp.kron(eye8, W_net)
    b2 = jnp.tile(b_net, PK).reshape(1, KD)
    k0 = lax.broadcasted_iota(jnp.int32, (128, KD), 0)
    c0 = lax.broadcasted_iota(jnp.int32, (128, KD), 1)
    r2 = ((k0 // IN_DIM == c0 // (IN_DIM * OUT_DIM))
          & (k0 % IN_DIM == (c0 % (IN_DIM * OUT_DIM)) // OUT_DIM)
          ).astype(jnp.float32)
    s0 = lax.broadcasted_iota(jnp.int32, (KD, 128), 0)
    s1 = lax.broadcasted_iota(jnp.int32, (KD, 128), 1)
    s2 = ((s0 // (IN_DIM * OUT_DIM) == s1 // OUT_DIM)
          & (s0 % OUT_DIM == s1 % OUT_DIM)).astype(jnp.float32)

    x_src = _sc_gather()(node_feat, src_p)
    msg = _msg_call(edge_feat.reshape(E8, 128), x_src.reshape(E8_PAD, 128),
                    w2, b2, r2, s2)
    agg_parts, cnt_parts = _sc_scatter()(
        msg.reshape(E_PAD, OUT_DIM), dst_p,
        jnp.zeros((ZROWS, OUT_DIM), jnp.float32),
        jnp.ones((HALF, OUT_DIM), jnp.float32),
    )

    wr2 = jnp.kron(eye8, W_root)
    m0 = lax.broadcasted_iota(jnp.int32, (128, 128), 0)
    m1 = lax.broadcasted_iota(jnp.int32, (128, 128), 1)
    m = (m0 % OUT_DIM == m1 % OUT_DIM).astype(jnp.float32)
    out = _final_call(
        node_feat.reshape(N8, 128), agg_parts.reshape(NC, N8, 128),
        cnt_parts.reshape(NC, N8, 128), wr2, m,
        jnp.tile(root_bias, PK).reshape(1, 128),
        jnp.tile(bn_gamma, PK).reshape(1, 128),
        jnp.tile(bn_beta, PK).reshape(1, 128),
    )
    return (out.reshape(N, OUT_DIM), edge_index, edge_feat)
